# baseline (speedup 1.0000x reference)
import functools

import jax
import jax.numpy as jnp
from jax import lax
from jax.experimental import pallas as pl
from jax.experimental.pallas import tpu as pltpu
from jax.experimental.pallas import tpu_sc as plsc

N_NODES = 100000
N_EDGES = 6400000

NC = 2
NS = 16
NW = NC * NS
CHUNK = 512
G = 2
GD = 8
KCH = 392
EPAD = NW * KCH * CHUNK
NPAD = 100352
RPS = NPAD // NS
RB = 2048


def _sc_mesh():
    return plsc.VectorSubcoreMesh(core_axis_name="c", subcore_axis_name="s")


_SC_PARAMS = pltpu.CompilerParams(use_tc_tiling_on_sc=False)


@functools.partial(
    pl.kernel,
    out_type=jax.ShapeDtypeStruct((NC, NPAD), jnp.float32),
    mesh=_sc_mesh(),
    compiler_params=_SC_PARAMS,
    scratch_types=[
        pltpu.VMEM((GD, CHUNK), jnp.int32),
        pltpu.VMEM((CHUNK,), jnp.float32),
        pltpu.VMEM_SHARED((NPAD,), jnp.float32),
        pltpu.SemaphoreType.DMA,
    ],
)
def _sc_degree(dstr, zeros1, out, dst_v, ones_v, deg_sp, ssem):
    c = lax.axis_index("c")
    s = lax.axis_index("s")
    wid = c * NS + s
    rsl = pl.ds(s * RPS, RPS)
    pltpu.sync_copy(zeros1.at[rsl], deg_sp.at[rsl])
    for i in range(CHUNK // 16):
        ones_v[pl.ds(i * 16, 16)] = jnp.ones((16,), jnp.float32)
    plsc.subcore_barrier()

    def body(o, carry):
        pltpu.sync_copy(dstr.at[wid, pl.ds(o * GD, GD)], dst_v)
        descs = [
            pltpu.async_copy(ones_v, deg_sp.at[dst_v.at[b]], ssem, add=True)
            for b in range(GD)
        ]
        for d_ in descs:
            d_.wait()
        return carry

    lax.fori_loop(0, KCH // GD, body, 0)
    plsc.subcore_barrier()
    pltpu.sync_copy(deg_sp.at[rsl], out.at[c, rsl])


def _make_sc_agg(D):
    @functools.partial(
        pl.kernel,
        out_type=jax.ShapeDtypeStruct((NC, NPAD, D), jnp.float32),
        mesh=_sc_mesh(),
        compiler_params=_SC_PARAMS,
        scratch_types=[
            pltpu.VMEM((G, CHUNK), jnp.int32),
            pltpu.VMEM((G, CHUNK), jnp.int32),
            pltpu.VMEM((G, CHUNK, D), jnp.float32),
            pltpu.VMEM_SHARED((NPAD, D), jnp.float32),
            pltpu.SemaphoreType.DMA,
            pltpu.SemaphoreType.DMA,
        ],
    )
    def sc_agg(y, srcr, dstr, zeros, out, src_v, dst_v, rows_v, agg_sp, gsem, ssem):
        c = lax.axis_index("c")
        s = lax.axis_index("s")
        wid = c * NS + s
        rsl = pl.ds(s * RPS, RPS)
        pltpu.sync_copy(zeros.at[rsl], agg_sp.at[rsl])
        plsc.subcore_barrier()

        def body(o, carry):
            pltpu.sync_copy(srcr.at[wid, pl.ds(o * G, G)], src_v)
            pltpu.sync_copy(dstr.at[wid, pl.ds(o * G, G)], dst_v)
            gd = [
                pltpu.async_copy(y.at[src_v.at[b]], rows_v.at[b], gsem)
                for b in range(G)
            ]
            sd = []
            for b in range(G):
                gd[b].wait()
                sd.append(
                    pltpu.async_copy(
                        rows_v.at[b], agg_sp.at[dst_v.at[b]], ssem, add=True
                    )
                )
            for d_ in sd:
                d_.wait()
            return carry

        lax.fori_loop(0, KCH // G, body, 0)
        plsc.subcore_barrier()
        pltpu.sync_copy(agg_sp.at[rsl], out.at[c, rsl])

    return sc_agg


_sc_agg16 = _make_sc_agg(16)


def _tc1_body(degp, x16, dinv_o, xs_o):
    deg = degp[0, :] + degp[1, :] + 1.0
    dinv = lax.rsqrt(deg)
    dinv_o[...] = dinv
    xs_o[...] = x16[...] * dinv[:, None]


def _tc1(degp, x16):
    return pl.pallas_call(
        _tc1_body,
        grid=(NPAD // RB,),
        in_specs=[
            pl.BlockSpec((NC, RB), lambda i: (0, i)),
            pl.BlockSpec((RB, 16), lambda i: (i, 0)),
        ],
        out_specs=[
            pl.BlockSpec((RB,), lambda i: (i,)),
            pl.BlockSpec((RB, 16), lambda i: (i, 0)),
        ],
        out_shape=[
            jax.ShapeDtypeStruct((NPAD,), jnp.float32),
            jax.ShapeDtypeStruct((NPAD, 16), jnp.float32),
        ],
    )(degp, x16)


def _tc2_body(aggp, xs, dinv, w1, b1, hs_o):
    dv = dinv[...][:, None]
    t = (aggp[0] + aggp[1] + xs[...]) * dv
    h1 = jnp.maximum(jnp.dot(t, w1[...]) + b1[...], 0.0)
    hs_o[...] = h1 * dv


def _tc2(aggp, xs, dinv, w1p, b1):
    return pl.pallas_call(
        _tc2_body,
        grid=(NPAD // RB,),
        in_specs=[
            pl.BlockSpec((NC, RB, 16), lambda i: (0, i, 0)),
            pl.BlockSpec((RB, 16), lambda i: (i, 0)),
            pl.BlockSpec((RB,), lambda i: (i,)),
            pl.BlockSpec((16, 16), lambda i: (0, 0)),
            pl.BlockSpec((16,), lambda i: (0,)),
        ],
        out_specs=pl.BlockSpec((RB, 16), lambda i: (i, 0)),
        out_shape=jax.ShapeDtypeStruct((NPAD, 16), jnp.float32),
    )(aggp, xs, dinv, w1p, b1)


def _tc3_body(aggp, hs, dinv, w2, b2, out_o):
    t = (aggp[0] + aggp[1] + hs[...]) * dinv[...][:, None]
    h2 = jnp.dot(t, w2[...]) + b2[...]
    m = jnp.max(h2, axis=1, keepdims=True)
    e = jnp.exp(h2 - m)
    lse = jnp.log(jnp.sum(e, axis=1, keepdims=True))
    out_o[...] = h2 - m - lse


def _tc3(aggp, hs, dinv, w2, b2):
    return pl.pallas_call(
        _tc3_body,
        grid=(NPAD // RB,),
        in_specs=[
            pl.BlockSpec((NC, RB, 16), lambda i: (0, i, 0)),
            pl.BlockSpec((RB, 16), lambda i: (i, 0)),
            pl.BlockSpec((RB,), lambda i: (i,)),
            pl.BlockSpec((16, 7), lambda i: (0, 0)),
            pl.BlockSpec((7,), lambda i: (0,)),
        ],
        out_specs=pl.BlockSpec((RB, 7), lambda i: (i, 0)),
        out_shape=jax.ShapeDtypeStruct((NPAD, 7), jnp.float32),
    )(aggp, hs, dinv, w2, b2)


def kernel(x, edge_index, W1, b1, W2, b2):
    epad = EPAD - N_EDGES
    src = jnp.concatenate(
        [edge_index[0], jnp.full((epad,), N_NODES, jnp.int32)]
    ).reshape(NW, KCH, CHUNK)
    dst = jnp.concatenate(
        [edge_index[1], jnp.full((epad,), N_NODES, jnp.int32)]
    ).reshape(NW, KCH, CHUNK)

    x16 = jnp.pad(x, ((0, NPAD - N_NODES), (0, 13)))
    w1p = jnp.pad(W1, ((0, 13), (0, 0)))
    z1 = jnp.zeros((NPAD,), jnp.float32)
    z16 = jnp.zeros((NPAD, 16), jnp.float32)

    degp = _sc_degree(dst, z1)
    dinv, xs = _tc1(degp, x16)
    agg1p = _sc_agg16(xs, src, dst, z16)
    hs = _tc2(agg1p, xs, dinv, w1p, b1)
    agg2p = _sc_agg16(hs, src, dst, z16)
    out = _tc3(agg2p, hs, dinv, W2, b2)
    return out[:N_NODES]


# speedup vs baseline: 1.1002x; 1.1002x over previous
import functools

import jax
import jax.numpy as jnp
from jax import lax
from jax.experimental import pallas as pl
from jax.experimental.pallas import tpu as pltpu
from jax.experimental.pallas import tpu_sc as plsc

N_NODES = 100000
N_EDGES = 6400000

NC = 2
NS = 16
NW = NC * NS
CHUNK = 256
G = 2
GD = 8
KCH = 784
NBLK = KCH // G
EPAD = NW * KCH * CHUNK
NPAD = 100352
RPS = NPAD // NS
RB = 2048


def _sc_mesh():
    return plsc.VectorSubcoreMesh(core_axis_name="c", subcore_axis_name="s")


_SC_PARAMS = pltpu.CompilerParams(use_tc_tiling_on_sc=False)


@functools.partial(
    pl.kernel,
    out_type=jax.ShapeDtypeStruct((NC, NPAD), jnp.float32),
    mesh=_sc_mesh(),
    compiler_params=_SC_PARAMS,
    scratch_types=[
        pltpu.VMEM((GD, CHUNK), jnp.int32),
        pltpu.VMEM((CHUNK,), jnp.float32),
        pltpu.VMEM_SHARED((NPAD,), jnp.float32),
        pltpu.SemaphoreType.DMA,
    ],
)
def _sc_degree(dstr, zeros1, out, dst_v, ones_v, deg_sp, ssem):
    c = lax.axis_index("c")
    s = lax.axis_index("s")
    wid = c * NS + s
    rsl = pl.ds(s * RPS, RPS)
    pltpu.sync_copy(zeros1.at[rsl], deg_sp.at[rsl])
    for i in range(CHUNK // 16):
        ones_v[pl.ds(i * 16, 16)] = jnp.ones((16,), jnp.float32)
    plsc.subcore_barrier()

    def body(o, carry):
        pltpu.sync_copy(dstr.at[wid, pl.ds(o * GD, GD)], dst_v)
        descs = [
            pltpu.async_copy(ones_v, deg_sp.at[dst_v.at[b]], ssem, add=True)
            for b in range(GD)
        ]
        for d_ in descs:
            d_.wait()
        return carry

    lax.fori_loop(0, KCH // GD, body, 0)
    plsc.subcore_barrier()
    pltpu.sync_copy(deg_sp.at[rsl], out.at[c, rsl])


def _make_sc_agg(D):
    @functools.partial(
        pl.kernel,
        out_type=jax.ShapeDtypeStruct((NC, NPAD, D), jnp.float32),
        mesh=_sc_mesh(),
        compiler_params=_SC_PARAMS,
        scratch_types=[
            pltpu.VMEM((G, CHUNK), jnp.int32),
            pltpu.VMEM((G, CHUNK), jnp.int32),
            pltpu.VMEM((G, CHUNK), jnp.int32),
            pltpu.VMEM((G, CHUNK), jnp.int32),
            pltpu.VMEM((G, CHUNK, D), jnp.float32),
            pltpu.VMEM((G, CHUNK, D), jnp.float32),
            pltpu.VMEM_SHARED((NPAD, D), jnp.float32),
            pltpu.SemaphoreType.DMA,
            pltpu.SemaphoreType.DMA,
            pltpu.SemaphoreType.DMA,
            pltpu.SemaphoreType.DMA,
        ],
    )
    def sc_agg(y, srcr, dstr, zeros, out,
               srcA, dstA, srcB, dstB, rowsA, rowsB, agg_sp,
               gsemA, gsemB, ssemA, ssemB):
        c = lax.axis_index("c")
        s = lax.axis_index("s")
        wid = c * NS + s
        rsl = pl.ds(s * RPS, RPS)
        pltpu.sync_copy(zeros.at[rsl], agg_sp.at[rsl])
        plsc.subcore_barrier()

        def load_idx(t, sv, dv):
            pltpu.sync_copy(srcr.at[wid, pl.ds(t * G, G)], sv)
            pltpu.sync_copy(dstr.at[wid, pl.ds(t * G, G)], dv)

        def issue_gathers(sv, rv, gsem):
            for b in range(G):
                pltpu.async_copy(y.at[sv.at[b]], rv.at[b], gsem)

        def wait_gathers(sv, rv, gsem):
            for b in range(G):
                pltpu.make_async_copy(y.at[sv.at[b]], rv.at[b], gsem).wait()

        def issue_scatters(dv, rv, ssem):
            for b in range(G):
                pltpu.async_copy(rv.at[b], agg_sp.at[dv.at[b]], ssem, add=True)

        def wait_scatters(dv, rv, ssem):
            for b in range(G):
                pltpu.make_async_copy(rv.at[b], agg_sp.at[dv.at[b]], ssem).wait()

        load_idx(0, srcA, dstA)
        issue_gathers(srcA, rowsA, gsemA)
        load_idx(1, srcB, dstB)
        issue_gathers(srcB, rowsB, gsemB)

        def half(t_next, sv, dv, rv, gsem, ssem):
            wait_gathers(sv, rv, gsem)
            issue_scatters(dv, rv, ssem)
            wait_scatters(dv, rv, ssem)
            load_idx(t_next, sv, dv)
            issue_gathers(sv, rv, gsem)

        def body(tt, carry):
            half(2 * tt + 2, srcA, dstA, rowsA, gsemA, ssemA)
            half(2 * tt + 3, srcB, dstB, rowsB, gsemB, ssemB)
            return carry

        lax.fori_loop(0, NBLK // 2 - 1, body, 0)

        wait_gathers(srcA, rowsA, gsemA)
        issue_scatters(dstA, rowsA, ssemA)
        wait_gathers(srcB, rowsB, gsemB)
        issue_scatters(dstB, rowsB, ssemB)
        wait_scatters(dstA, rowsA, ssemA)
        wait_scatters(dstB, rowsB, ssemB)

        plsc.subcore_barrier()
        pltpu.sync_copy(agg_sp.at[rsl], out.at[c, rsl])

    return sc_agg


_sc_agg16 = _make_sc_agg(16)


def _tc1_body(degp, x16, dinv_o, xs_o):
    deg = degp[0, :] + degp[1, :] + 1.0
    dinv = lax.rsqrt(deg)
    dinv_o[...] = dinv
    xs_o[...] = x16[...] * dinv[:, None]


def _tc1(degp, x16):
    return pl.pallas_call(
        _tc1_body,
        grid=(NPAD // RB,),
        in_specs=[
            pl.BlockSpec((NC, RB), lambda i: (0, i)),
            pl.BlockSpec((RB, 16), lambda i: (i, 0)),
        ],
        out_specs=[
            pl.BlockSpec((RB,), lambda i: (i,)),
            pl.BlockSpec((RB, 16), lambda i: (i, 0)),
        ],
        out_shape=[
            jax.ShapeDtypeStruct((NPAD,), jnp.float32),
            jax.ShapeDtypeStruct((NPAD, 16), jnp.float32),
        ],
    )(degp, x16)


def _tc2_body(aggp, xs, dinv, w1, b1, hs_o):
    dv = dinv[...][:, None]
    t = (aggp[0] + aggp[1] + xs[...]) * dv
    h1 = jnp.maximum(jnp.dot(t, w1[...]) + b1[...], 0.0)
    hs_o[...] = h1 * dv


def _tc2(aggp, xs, dinv, w1p, b1):
    return pl.pallas_call(
        _tc2_body,
        grid=(NPAD // RB,),
        in_specs=[
            pl.BlockSpec((NC, RB, 16), lambda i: (0, i, 0)),
            pl.BlockSpec((RB, 16), lambda i: (i, 0)),
            pl.BlockSpec((RB,), lambda i: (i,)),
            pl.BlockSpec((16, 16), lambda i: (0, 0)),
            pl.BlockSpec((16,), lambda i: (0,)),
        ],
        out_specs=pl.BlockSpec((RB, 16), lambda i: (i, 0)),
        out_shape=jax.ShapeDtypeStruct((NPAD, 16), jnp.float32),
    )(aggp, xs, dinv, w1p, b1)


def _tc3_body(aggp, hs, dinv, w2, b2, out_o):
    t = (aggp[0] + aggp[1] + hs[...]) * dinv[...][:, None]
    h2 = jnp.dot(t, w2[...]) + b2[...]
    m = jnp.max(h2, axis=1, keepdims=True)
    e = jnp.exp(h2 - m)
    lse = jnp.log(jnp.sum(e, axis=1, keepdims=True))
    out_o[...] = h2 - m - lse


def _tc3(aggp, hs, dinv, w2, b2):
    return pl.pallas_call(
        _tc3_body,
        grid=(NPAD // RB,),
        in_specs=[
            pl.BlockSpec((NC, RB, 16), lambda i: (0, i, 0)),
            pl.BlockSpec((RB, 16), lambda i: (i, 0)),
            pl.BlockSpec((RB,), lambda i: (i,)),
            pl.BlockSpec((16, 7), lambda i: (0, 0)),
            pl.BlockSpec((7,), lambda i: (0,)),
        ],
        out_specs=pl.BlockSpec((RB, 7), lambda i: (i, 0)),
        out_shape=jax.ShapeDtypeStruct((NPAD, 7), jnp.float32),
    )(aggp, hs, dinv, w2, b2)


def kernel(x, edge_index, W1, b1, W2, b2):
    epad = EPAD - N_EDGES
    src = jnp.concatenate(
        [edge_index[0], jnp.full((epad,), N_NODES, jnp.int32)]
    ).reshape(NW, KCH, CHUNK)
    dst = jnp.concatenate(
        [edge_index[1], jnp.full((epad,), N_NODES, jnp.int32)]
    ).reshape(NW, KCH, CHUNK)

    x16 = jnp.pad(x, ((0, NPAD - N_NODES), (0, 13)))
    w1p = jnp.pad(W1, ((0, 13), (0, 0)))
    z1 = jnp.zeros((NPAD,), jnp.float32)
    z16 = jnp.zeros((NPAD, 16), jnp.float32)

    degp = _sc_degree(dst, z1)
    dinv, xs = _tc1(degp, x16)
    agg1p = _sc_agg16(xs, src, dst, z16)
    hs = _tc2(agg1p, xs, dinv, w1p, b1)
    agg2p = _sc_agg16(hs, src, dst, z16)
    out = _tc3(agg2p, hs, dinv, W2, b2)
    return out[:N_NODES]


# speedup vs baseline: 1.4466x; 1.3149x over previous
import functools

import jax
import jax.numpy as jnp
from jax import lax
from jax.experimental import pallas as pl
from jax.experimental.pallas import tpu as pltpu
from jax.experimental.pallas import tpu_sc as plsc

N_NODES = 100000
N_EDGES = 6400000

NC = 2
NS = 16
NW = NC * NS
CHUNK = 400
G = 2
GD = 10
KCH = 500
NBLK = KCH // G
NPAD = 100352
RPS = NPAD // NS
RB = 2048


def _sc_mesh():
    return plsc.VectorSubcoreMesh(core_axis_name="c", subcore_axis_name="s")


_SC_PARAMS = pltpu.CompilerParams(use_tc_tiling_on_sc=False)


@functools.partial(
    pl.kernel,
    out_type=jax.ShapeDtypeStruct((NC, NPAD), jnp.float32),
    mesh=_sc_mesh(),
    compiler_params=_SC_PARAMS,
    scratch_types=[
        pltpu.VMEM((GD, CHUNK), jnp.int32),
        pltpu.VMEM((CHUNK,), jnp.float32),
        pltpu.VMEM_SHARED((NPAD,), jnp.float32),
        pltpu.SemaphoreType.DMA,
    ],
)
def _sc_degree(er, zeros1, out, dst_v, ones_v, deg_sp, ssem):
    c = lax.axis_index("c")
    s = lax.axis_index("s")
    wid = c * NS + s
    rsl = pl.ds(s * RPS, RPS)
    pltpu.sync_copy(zeros1.at[rsl], deg_sp.at[rsl])
    for i in range(CHUNK // 16):
        ones_v[pl.ds(i * 16, 16)] = jnp.ones((16,), jnp.float32)
    plsc.subcore_barrier()

    def body(o, carry):
        pltpu.sync_copy(er.at[1, wid, pl.ds(o * GD, GD)], dst_v)
        descs = [
            pltpu.async_copy(ones_v, deg_sp.at[dst_v.at[b]], ssem, add=True)
            for b in range(GD)
        ]
        for d_ in descs:
            d_.wait()
        return carry

    lax.fori_loop(0, KCH // GD, body, 0)
    plsc.subcore_barrier()
    pltpu.sync_copy(deg_sp.at[rsl], out.at[c, rsl])


def _make_sc_agg(D):
    @functools.partial(
        pl.kernel,
        out_type=jax.ShapeDtypeStruct((NC, NPAD, D), jnp.float32),
        mesh=_sc_mesh(),
        compiler_params=_SC_PARAMS,
        scratch_types=[
            pltpu.VMEM((G, CHUNK), jnp.int32),
            pltpu.VMEM((G, CHUNK), jnp.int32),
            pltpu.VMEM((G, CHUNK), jnp.int32),
            pltpu.VMEM((G, CHUNK), jnp.int32),
            pltpu.VMEM((G, CHUNK, D), jnp.float32),
            pltpu.VMEM((G, CHUNK, D), jnp.float32),
            pltpu.VMEM_SHARED((NPAD, D), jnp.float32),
            pltpu.SemaphoreType.DMA,
            pltpu.SemaphoreType.DMA,
            pltpu.SemaphoreType.DMA,
            pltpu.SemaphoreType.DMA,
        ],
    )
    def sc_agg(y, er, zeros, out,
               srcA, dstA, srcB, dstB, rowsA, rowsB, agg_sp,
               gsemA, gsemB, ssemA, ssemB):
        c = lax.axis_index("c")
        s = lax.axis_index("s")
        wid = c * NS + s
        rsl = pl.ds(s * RPS, RPS)
        pltpu.sync_copy(zeros.at[rsl], agg_sp.at[rsl])
        plsc.subcore_barrier()

        def load_idx(t, sv, dv):
            pltpu.sync_copy(er.at[0, wid, pl.ds(t * G, G)], sv)
            pltpu.sync_copy(er.at[1, wid, pl.ds(t * G, G)], dv)

        def issue_gathers(sv, rv, gsem):
            for b in range(G):
                pltpu.async_copy(y.at[sv.at[b]], rv.at[b], gsem)

        def wait_gathers(sv, rv, gsem):
            for b in range(G):
                pltpu.make_async_copy(y.at[sv.at[b]], rv.at[b], gsem).wait()

        def issue_scatters(dv, rv, ssem):
            for b in range(G):
                pltpu.async_copy(rv.at[b], agg_sp.at[dv.at[b]], ssem, add=True)

        def wait_scatters(dv, rv, ssem):
            for b in range(G):
                pltpu.make_async_copy(rv.at[b], agg_sp.at[dv.at[b]], ssem).wait()

        load_idx(0, srcA, dstA)
        issue_gathers(srcA, rowsA, gsemA)
        load_idx(1, srcB, dstB)
        issue_gathers(srcB, rowsB, gsemB)

        def half(t_next, sv, dv, rv, gsem, ssem):
            wait_gathers(sv, rv, gsem)
            issue_scatters(dv, rv, ssem)
            wait_scatters(dv, rv, ssem)
            load_idx(t_next, sv, dv)
            issue_gathers(sv, rv, gsem)

        def body(tt, carry):
            half(2 * tt + 2, srcA, dstA, rowsA, gsemA, ssemA)
            half(2 * tt + 3, srcB, dstB, rowsB, gsemB, ssemB)
            return carry

        lax.fori_loop(0, NBLK // 2 - 1, body, 0)

        wait_gathers(srcA, rowsA, gsemA)
        issue_scatters(dstA, rowsA, ssemA)
        wait_gathers(srcB, rowsB, gsemB)
        issue_scatters(dstB, rowsB, ssemB)
        wait_scatters(dstA, rowsA, ssemA)
        wait_scatters(dstB, rowsB, ssemB)

        plsc.subcore_barrier()
        pltpu.sync_copy(agg_sp.at[rsl], out.at[c, rsl])

    return sc_agg


_sc_agg16 = _make_sc_agg(16)


def _tc1_body(degp, x16, dinv_o, xs_o):
    deg = degp[0, :] + degp[1, :] + 1.0
    dinv = lax.rsqrt(deg)
    dinv_o[...] = dinv
    xs_o[...] = x16[...] * dinv[:, None]


def _tc1(degp, x16):
    return pl.pallas_call(
        _tc1_body,
        grid=(NPAD // RB,),
        in_specs=[
            pl.BlockSpec((NC, RB), lambda i: (0, i)),
            pl.BlockSpec((RB, 16), lambda i: (i, 0)),
        ],
        out_specs=[
            pl.BlockSpec((RB,), lambda i: (i,)),
            pl.BlockSpec((RB, 16), lambda i: (i, 0)),
        ],
        out_shape=[
            jax.ShapeDtypeStruct((NPAD,), jnp.float32),
            jax.ShapeDtypeStruct((NPAD, 16), jnp.float32),
        ],
    )(degp, x16)


def _tc2_body(aggp, xs, dinv, w1, b1, hs_o):
    dv = dinv[...][:, None]
    t = (aggp[0] + aggp[1] + xs[...]) * dv
    h1 = jnp.maximum(jnp.dot(t, w1[...]) + b1[...], 0.0)
    hs_o[...] = h1 * dv


def _tc2(aggp, xs, dinv, w1p, b1):
    return pl.pallas_call(
        _tc2_body,
        grid=(NPAD // RB,),
        in_specs=[
            pl.BlockSpec((NC, RB, 16), lambda i: (0, i, 0)),
            pl.BlockSpec((RB, 16), lambda i: (i, 0)),
            pl.BlockSpec((RB,), lambda i: (i,)),
            pl.BlockSpec((16, 16), lambda i: (0, 0)),
            pl.BlockSpec((16,), lambda i: (0,)),
        ],
        out_specs=pl.BlockSpec((RB, 16), lambda i: (i, 0)),
        out_shape=jax.ShapeDtypeStruct((NPAD, 16), jnp.float32),
    )(aggp, xs, dinv, w1p, b1)


def _tc3_body(aggp, hs, dinv, w2, b2, out_o):
    t = (aggp[0] + aggp[1] + hs[...]) * dinv[...][:, None]
    h2 = jnp.dot(t, w2[...]) + b2[...]
    m = jnp.max(h2, axis=1, keepdims=True)
    e = jnp.exp(h2 - m)
    lse = jnp.log(jnp.sum(e, axis=1, keepdims=True))
    out_o[...] = h2 - m - lse


def _tc3(aggp, hs, dinv, w2, b2):
    return pl.pallas_call(
        _tc3_body,
        grid=(NPAD // RB,),
        in_specs=[
            pl.BlockSpec((NC, RB, 16), lambda i: (0, i, 0)),
            pl.BlockSpec((RB, 16), lambda i: (i, 0)),
            pl.BlockSpec((RB,), lambda i: (i,)),
            pl.BlockSpec((16, 7), lambda i: (0, 0)),
            pl.BlockSpec((7,), lambda i: (0,)),
        ],
        out_specs=pl.BlockSpec((RB, 7), lambda i: (i, 0)),
        out_shape=jax.ShapeDtypeStruct((NPAD, 7), jnp.float32),
    )(aggp, hs, dinv, w2, b2)


def kernel(x, edge_index, W1, b1, W2, b2):
    er = edge_index.reshape(2, NW, KCH, CHUNK)

    x16 = jnp.pad(x, ((0, NPAD - N_NODES), (0, 13)))
    w1p = jnp.pad(W1, ((0, 13), (0, 0)))
    z1 = jnp.zeros((NPAD,), jnp.float32)
    z16 = jnp.zeros((NPAD, 16), jnp.float32)

    degp = _sc_degree(er, z1)
    dinv, xs = _tc1(degp, x16)
    agg1p = _sc_agg16(xs, er, z16)
    hs = _tc2(agg1p, xs, dinv, w1p, b1)
    agg2p = _sc_agg16(hs, er, z16)
    out = _tc3(agg2p, hs, dinv, W2, b2)
    return out[:N_NODES]
